# Initial kernel scaffold; baseline (speedup 1.0000x reference)
import jax
import jax.numpy as jnp
from jax.experimental import pallas as pl


def kernel(x, edge_index, mlp_W, mlp_b, mlp_bn_g, mlp_bn_b, Wg0, Wcon0, Wdis0, Wself0, bself0, gW0, gb0, bn0_g, bn0_b, Wg1, Wcon1, Wdis1, Wself1, bself1, gW1, gb1, cls_W, cls_b):
    raise NotImplementedError("write your pallas kernel here")


# speedup vs baseline: 7.4433x; 7.4433x over previous
import functools

import jax
import jax.numpy as jnp
from jax import lax
from jax.experimental import pallas as pl
from jax.experimental.pallas import tpu as pltpu
from jax.experimental.pallas import tpu_sc as plsc

N = 10000
D = 128
H = 128
C = 64
EPS = 1e-5
E1 = 2.718281828459045

NPAD = 10240
L = 16
NCORE = 2
NSUB = 16
NWORK = NCORE * NSUB
CHUNK = 128
ETOT = 320000 + N
CPW = -(-ETOT // (NWORK * CHUNK))
EPAD = NWORK * CPW * CHUNK
CPS = EPAD // (NSUB * CHUNK)
NPT = NPAD // NSUB


def _lin_body(x_ref, w_ref, b_ref, s_ref, t_ref, o_ref, *, relu):
    y = jnp.dot(x_ref[...], w_ref[...], preferred_element_type=jnp.float32)
    y = (y + b_ref[...]) * s_ref[...] + t_ref[...]
    if relu:
        y = jnp.maximum(y, 0.0)
    o_ref[...] = y


def _lin(x, w, b, s, t, relu):
    n, k = x.shape
    m = w.shape[1]
    br = 1024
    return pl.pallas_call(
        functools.partial(_lin_body, relu=relu),
        grid=(n // br,),
        in_specs=[
            pl.BlockSpec((br, k), lambda i: (i, 0)),
            pl.BlockSpec((k, m), lambda i: (0, 0)),
            pl.BlockSpec((1, m), lambda i: (0, 0)),
            pl.BlockSpec((1, m), lambda i: (0, 0)),
            pl.BlockSpec((1, m), lambda i: (0, 0)),
        ],
        out_specs=pl.BlockSpec((br, m), lambda i: (i, 0)),
        out_shape=jax.ShapeDtypeStruct((n, m), jnp.float32),
    )(x, w, b, s, t)


def _tail_body(oc_ref, od_ref, osf_ref, hin_ref, gw_ref, gb_ref, bs_ref,
               bt_ref, o_ref, *, bnrelu):
    oc = oc_ref[...]
    od = od_ref[...]
    osf = osf_ref[...]

    def logit(k):
        a = gw_ref[k:k + 1, 0:H]
        b = gw_ref[k:k + 1, H:2 * H]
        cc = gw_ref[k:k + 1, 2 * H:3 * H]
        return (jnp.sum(oc * a + od * b + osf * cc, axis=1, keepdims=True)
                + gb_ref[0:1, k:k + 1])

    z0 = logit(0)
    z1 = logit(1)
    z2 = logit(2)
    m = jnp.maximum(jnp.maximum(z0, z1), z2)
    e0 = jnp.exp(z0 - m)
    e1 = jnp.exp(z1 - m)
    e2 = jnp.exp(z2 - m)
    den = e0 + e1 + e2
    y = (e0 * oc + e1 * od + e2 * osf) / den
    if bnrelu:
        y = jnp.maximum(y * bs_ref[...] + bt_ref[...], 0.0)
    o_ref[...] = y + hin_ref[...]


def _tail(oc, od, osf, hin, gw, gb128, bs, bt, bnrelu):
    n = oc.shape[0]
    br = 1024
    blk = lambda r, c: pl.BlockSpec((r, c), lambda i: (i, 0))
    fix = lambda r, c: pl.BlockSpec((r, c), lambda i: (0, 0))
    return pl.pallas_call(
        functools.partial(_tail_body, bnrelu=bnrelu),
        grid=(n // br,),
        in_specs=[
            blk(br, H), blk(br, H), blk(br, H), blk(br, H),
            fix(3, 3 * H), fix(1, H), fix(1, H), fix(1, H),
        ],
        out_specs=blk(br, H),
        out_shape=jax.ShapeDtypeStruct((n, H), jnp.float32),
    )(oc, od, osf, hin, gw, gb128, bs, bt)


def _newton_rsqrt(t):
    i32 = lax.bitcast_convert_type(t, jnp.int32)
    i32 = 0x5F3759DF - lax.shift_right_logical(i32, 1)
    y = lax.bitcast_convert_type(i32, jnp.float32)
    y = y * (1.5 - 0.5 * t * y * y)
    y = y * (1.5 - 0.5 * t * y * y)
    y = y * (1.5 - 0.5 * t * y * y)
    return y


def _phase1_body(xg, row, col, es_out, dp_out,
                 ri, ci, rr, rc, ec, ed, sred, pack, dstage, dcon_sh, ddis_sh,
                 sem0, sem1):
    c = lax.axis_index("c")
    s = lax.axis_index("s")
    wid = c * NSUB + s
    z16 = jnp.zeros((L,), jnp.float32)
    for k in range(NPT // L):
        dstage[pl.ds(k * L, L)] = z16
    pltpu.sync_copy(dstage, dcon_sh.at[pl.ds(s * NPT, NPT)])
    pltpu.sync_copy(dstage, ddis_sh.at[pl.ds(s * NPT, NPT)])
    plsc.subcore_barrier()

    def chunk_body(ch, carry):
        base = (wid * CPW + ch) * CHUNK
        pltpu.sync_copy(row.at[pl.ds(base, CHUNK)], ri)
        pltpu.sync_copy(col.at[pl.ds(base, CHUNK)], ci)
        cp1 = pltpu.async_copy(xg.at[ri], rr, sem0)
        cp2 = pltpu.async_copy(xg.at[ci], rc, sem1)
        cp1.wait()
        cp2.wait()

        def grp_body(k, cy):
            for l in range(L):
                e = k * L + l
                acc = jnp.zeros((L,), jnp.float32)
                for j in range(H // L):
                    a = rr[e, pl.ds(j * L, L)]
                    b = rc[e, pl.ds(j * L, L)]
                    d = a - b
                    acc = acc + d * d
                sred[pl.ds(0, L)] = acc
                v = sred[pl.ds(0, L)] + sred[pl.ds(8, L)]
                sred[pl.ds(0, L)] = v
                v = v + sred[pl.ds(4, L)]
                sred[pl.ds(0, L)] = v
                v = v + sred[pl.ds(2, L)]
                sred[pl.ds(0, L)] = v
                v = v + sred[pl.ds(1, L)]
                pack[pl.ds(l, L)] = v
            t = pack[pl.ds(0, L)] + 1e-12
            g = t * _newton_rsqrt(t)
            sg = 1.0 / (1.0 + jnp.exp(g))
            ecv = jnp.exp(sg)
            ec[pl.ds(k * L, L)] = ecv
            ed[pl.ds(k * L, L)] = E1 / ecv
            return cy

        lax.fori_loop(0, CHUNK // L, grp_body, 0)
        pltpu.sync_copy(ec, es_out.at[0, pl.ds(base, CHUNK)])
        pltpu.sync_copy(ed, es_out.at[1, pl.ds(base, CHUNK)])
        pltpu.sync_copy(ec, dcon_sh.at[ri], add=True)
        pltpu.sync_copy(ed, ddis_sh.at[ri], add=True)
        return carry

    lax.fori_loop(0, CPW, chunk_body, 0)
    plsc.subcore_barrier()
    pltpu.sync_copy(dcon_sh.at[pl.ds(s * NPT, NPT)], dstage)
    pltpu.sync_copy(dstage, dp_out.at[c, 0, pl.ds(s * NPT, NPT)])
    pltpu.sync_copy(ddis_sh.at[pl.ds(s * NPT, NPT)], dstage)
    pltpu.sync_copy(dstage, dp_out.at[c, 1, pl.ds(s * NPT, NPT)])


_phase1 = functools.partial(
    pl.kernel,
    mesh=plsc.VectorSubcoreMesh(core_axis_name="c", subcore_axis_name="s"),
    out_type=(
        jax.ShapeDtypeStruct((2, EPAD), jnp.float32),
        jax.ShapeDtypeStruct((NCORE, 2, NPAD), jnp.float32),
    ),
    scratch_types=[
        pltpu.VMEM((CHUNK,), jnp.int32),
        pltpu.VMEM((CHUNK,), jnp.int32),
        pltpu.VMEM((CHUNK, H), jnp.float32),
        pltpu.VMEM((CHUNK, H), jnp.float32),
        pltpu.VMEM((CHUNK,), jnp.float32),
        pltpu.VMEM((CHUNK,), jnp.float32),
        pltpu.VMEM((32,), jnp.float32),
        pltpu.VMEM((48,), jnp.float32),
        pltpu.VMEM((NPT,), jnp.float32),
        pltpu.VMEM_SHARED((NPAD,), jnp.float32),
        pltpu.VMEM_SHARED((NPAD,), jnp.float32),
        pltpu.SemaphoreType.DMA,
        pltpu.SemaphoreType.DMA,
    ],
)(_phase1_body)


def _norm_body(xc_ref, xd_ref, dp_ref, oc_ref, od_ref):
    dcol = dp_ref[...]
    dcon = dcol[:, 0:1] + dcol[:, 2:3] + 1e-16
    ddis = dcol[:, 1:2] + dcol[:, 3:4] + 1e-16
    oc_ref[...] = xc_ref[...] / dcon
    od_ref[...] = xd_ref[...] / ddis


def _norm(xc, xd, dp_t):
    n = xc.shape[0]
    br = 1024
    blk = lambda c: pl.BlockSpec((br, c), lambda i: (i, 0))
    return pl.pallas_call(
        _norm_body,
        grid=(n // br,),
        in_specs=[blk(H), blk(H), blk(4)],
        out_specs=[blk(H), blk(H)],
        out_shape=(jax.ShapeDtypeStruct((n, H), jnp.float32),
                   jax.ShapeDtypeStruct((n, H), jnp.float32)),
    )(xc, xd, dp_t)


def _phase2_body(xall, row, col, esw, out2,
                 ri, ci, rows, ewb, oacc, sem0):
    c = lax.axis_index("c")
    s = lax.axis_index("s")

    def zrow(r, cy):
        for j in range(H // L):
            rows[r, pl.ds(j * L, L)] = jnp.zeros((L,), jnp.float32)
        return cy

    lax.fori_loop(0, CHUNK, zrow, 0)
    for b in range(NPT // CHUNK):
        pltpu.sync_copy(rows, oacc.at[pl.ds(s * NPT + b * CHUNK, CHUNK)])
    plsc.subcore_barrier()

    def chunk_body(ch, carry):
        base = (s * CPS + ch) * CHUNK
        pltpu.sync_copy(row.at[pl.ds(base, CHUNK)], ri)
        pltpu.sync_copy(col.at[pl.ds(base, CHUNK)], ci)
        pltpu.sync_copy(esw.at[c, pl.ds(base, CHUNK)], ewb)

        def offs(k, cy):
            ri[pl.ds(k * L, L)] = ri[pl.ds(k * L, L)] + c * NPAD
            return cy

        lax.fori_loop(0, CHUNK // L, offs, 0)
        pltpu.async_copy(xall.at[ri], rows, sem0).wait()

        def scale_body(e, cy):
            w = ewb[e, pl.ds(0, L)]
            for j in range(H // L):
                rows[e, pl.ds(j * L, L)] = rows[e, pl.ds(j * L, L)] * w
            return cy

        lax.fori_loop(0, CHUNK, scale_body, 0)
        pltpu.sync_copy(rows, oacc.at[ci], add=True)
        return carry

    lax.fori_loop(0, CPS, chunk_body, 0)
    plsc.subcore_barrier()
    for b in range(NPT // CHUNK):
        r0 = s * NPT + b * CHUNK
        pltpu.sync_copy(oacc.at[pl.ds(r0, CHUNK)], rows)
        pltpu.sync_copy(rows, out2.at[c, pl.ds(r0, CHUNK)])


_phase2 = functools.partial(
    pl.kernel,
    mesh=plsc.VectorSubcoreMesh(core_axis_name="c", subcore_axis_name="s"),
    out_type=jax.ShapeDtypeStruct((NCORE, NPAD, H), jnp.float32),
    scratch_types=[
        pltpu.VMEM((CHUNK,), jnp.int32),
        pltpu.VMEM((CHUNK,), jnp.int32),
        pltpu.VMEM((CHUNK, H), jnp.float32),
        pltpu.VMEM((CHUNK, L), jnp.float32),
        pltpu.VMEM_SHARED((NPAD, H), jnp.float32),
        pltpu.SemaphoreType.DMA,
    ],
)(_phase2_body)


def kernel(x, edge_index, mlp_W, mlp_b, mlp_bn_g, mlp_bn_b,
           Wg0, Wcon0, Wdis0, Wself0, bself0, gW0, gb0,
           bn0_g, bn0_b,
           Wg1, Wcon1, Wdis1, Wself1, bself1, gW1, gb1,
           cls_W, cls_b):
    f32 = jnp.float32
    x_pad = jnp.zeros((NPAD, D), f32).at[:N].set(x)
    sl = jnp.arange(N, dtype=jnp.int32)
    pad_e = jnp.full((EPAD - ETOT,), N, jnp.int32)
    rowp = jnp.concatenate([edge_index[0].astype(jnp.int32), sl, pad_e])
    colp = jnp.concatenate([edge_index[1].astype(jnp.int32), sl, pad_e])

    ones_h = jnp.ones((1, H), f32)
    zeros_h = jnp.zeros((1, H), f32)
    bn_sc = (mlp_bn_g / jnp.sqrt(1.0 + EPS)).reshape(1, H)
    h = _lin(x_pad, mlp_W.T, mlp_b.reshape(1, H), bn_sc,
             mlp_bn_b.reshape(1, H), True)

    def conv_layer(hh, Wg, Wcon, Wdis, Wself, bself, gW, gb, bs, bt, bnrelu):
        wall = jnp.concatenate([Wg.T, Wcon.T, Wdis.T, Wself.T], axis=1)
        ball = jnp.concatenate([jnp.zeros((3 * H,), f32), bself]).reshape(1, 4 * H)
        on4 = jnp.ones((1, 4 * H), f32)
        ze4 = jnp.zeros((1, 4 * H), f32)
        y4 = _lin(hh, wall, ball, on4, ze4, False)
        xg = y4[:, 0:H]
        xcon = y4[:, H:2 * H]
        xdis = y4[:, 2 * H:3 * H]
        osf = y4[:, 3 * H:4 * H]
        es_arr, dparts = _phase1(xg, rowp, colp)
        dp_t = jnp.moveaxis(dparts, 2, 0).reshape(NPAD, 4)
        xcp, xdp = _norm(xcon, xdis, dp_t)
        esw = jnp.broadcast_to(es_arr[:, :, None], (2, EPAD, L))
        xall = jnp.concatenate([xcp, xdp], axis=0)
        out2 = _phase2(xall, rowp, colp, esw)
        gb128 = jnp.zeros((1, H), f32).at[0, :3].set(gb)
        return _tail(out2[0], out2[1], osf, hh, gW, gb128, bs, bt, bnrelu)

    bn0_sc = (bn0_g / jnp.sqrt(1.0 + EPS)).reshape(1, H)
    h = conv_layer(h, Wg0, Wcon0, Wdis0, Wself0, bself0, gW0, gb0,
                   bn0_sc, bn0_b.reshape(1, H), True)
    h = conv_layer(h, Wg1, Wcon1, Wdis1, Wself1, bself1, gW1, gb1,
                   ones_h, zeros_h, False)

    ones_c = jnp.ones((1, C), f32)
    zeros_c = jnp.zeros((1, C), f32)
    logits = _lin(h, cls_W.T, cls_b.reshape(1, C), ones_c, zeros_c, False)
    return logits[:N]


# speedup vs baseline: 7.5277x; 1.0113x over previous
import functools

import jax
import jax.numpy as jnp
from jax import lax
from jax.experimental import pallas as pl
from jax.experimental.pallas import tpu as pltpu
from jax.experimental.pallas import tpu_sc as plsc

N = 10000
D = 128
H = 128
C = 64
EPS = 1e-5
E1 = 2.718281828459045

NPAD = 10240
L = 16
NCORE = 2
NSUB = 16
NWORK = NCORE * NSUB
CHUNK = 128
ETOT = 320000 + N
CPW = -(-ETOT // (NWORK * CHUNK))
EPAD = NWORK * CPW * CHUNK
CH2 = 64
CPS2 = EPAD // (NSUB * CH2)
NPT = NPAD // NSUB


def _lin_body(x_ref, w_ref, b_ref, s_ref, t_ref, o_ref, *, relu):
    y = jnp.dot(x_ref[...], w_ref[...], preferred_element_type=jnp.float32)
    y = (y + b_ref[...]) * s_ref[...] + t_ref[...]
    if relu:
        y = jnp.maximum(y, 0.0)
    o_ref[...] = y


def _lin(x, w, b, s, t, relu):
    n, k = x.shape
    m = w.shape[1]
    br = 1024
    return pl.pallas_call(
        functools.partial(_lin_body, relu=relu),
        grid=(n // br,),
        in_specs=[
            pl.BlockSpec((br, k), lambda i: (i, 0)),
            pl.BlockSpec((k, m), lambda i: (0, 0)),
            pl.BlockSpec((1, m), lambda i: (0, 0)),
            pl.BlockSpec((1, m), lambda i: (0, 0)),
            pl.BlockSpec((1, m), lambda i: (0, 0)),
        ],
        out_specs=pl.BlockSpec((br, m), lambda i: (i, 0)),
        out_shape=jax.ShapeDtypeStruct((n, m), jnp.float32),
    )(x, w, b, s, t)


def _tail_body(oc_ref, od_ref, osf_ref, hin_ref, gw_ref, gb_ref, bs_ref,
               bt_ref, o_ref, *, bnrelu):
    oc = oc_ref[...]
    od = od_ref[...]
    osf = osf_ref[...]

    def logit(k):
        a = gw_ref[k:k + 1, 0:H]
        b = gw_ref[k:k + 1, H:2 * H]
        cc = gw_ref[k:k + 1, 2 * H:3 * H]
        return (jnp.sum(oc * a + od * b + osf * cc, axis=1, keepdims=True)
                + gb_ref[0:1, k:k + 1])

    z0 = logit(0)
    z1 = logit(1)
    z2 = logit(2)
    m = jnp.maximum(jnp.maximum(z0, z1), z2)
    e0 = jnp.exp(z0 - m)
    e1 = jnp.exp(z1 - m)
    e2 = jnp.exp(z2 - m)
    den = e0 + e1 + e2
    y = (e0 * oc + e1 * od + e2 * osf) / den
    if bnrelu:
        y = jnp.maximum(y * bs_ref[...] + bt_ref[...], 0.0)
    o_ref[...] = y + hin_ref[...]


def _tail(oc, od, osf, hin, gw, gb128, bs, bt, bnrelu):
    n = oc.shape[0]
    br = 1024
    blk = lambda r, c: pl.BlockSpec((r, c), lambda i: (i, 0))
    fix = lambda r, c: pl.BlockSpec((r, c), lambda i: (0, 0))
    return pl.pallas_call(
        functools.partial(_tail_body, bnrelu=bnrelu),
        grid=(n // br,),
        in_specs=[
            blk(br, H), blk(br, H), blk(br, H), blk(br, H),
            fix(3, 3 * H), fix(1, H), fix(1, H), fix(1, H),
        ],
        out_specs=blk(br, H),
        out_shape=jax.ShapeDtypeStruct((n, H), jnp.float32),
    )(oc, od, osf, hin, gw, gb128, bs, bt)


def _newton_rsqrt(t):
    i32 = lax.bitcast_convert_type(t, jnp.int32)
    i32 = 0x5F3759DF - lax.shift_right_logical(i32, 1)
    y = lax.bitcast_convert_type(i32, jnp.float32)
    y = y * (1.5 - 0.5 * t * y * y)
    y = y * (1.5 - 0.5 * t * y * y)
    y = y * (1.5 - 0.5 * t * y * y)
    return y


def _phase1_body(xg, row, col, es_out, dp_out,
                 ri, ci, rr, rc, ec, ed, sred, pack, dstage, dcon_sh, ddis_sh,
                 sem0, sem1):
    c = lax.axis_index("c")
    s = lax.axis_index("s")
    wid = c * NSUB + s
    z16 = jnp.zeros((L,), jnp.float32)
    for k in range(NPT // L):
        dstage[pl.ds(k * L, L)] = z16
    pltpu.sync_copy(dstage, dcon_sh.at[pl.ds(s * NPT, NPT)])
    pltpu.sync_copy(dstage, ddis_sh.at[pl.ds(s * NPT, NPT)])
    plsc.subcore_barrier()

    def chunk_body(ch, carry):
        base = (wid * CPW + ch) * CHUNK
        pltpu.sync_copy(row.at[pl.ds(base, CHUNK)], ri)
        pltpu.sync_copy(col.at[pl.ds(base, CHUNK)], ci)
        cp1 = pltpu.async_copy(xg.at[ri], rr, sem0)
        cp2 = pltpu.async_copy(xg.at[ci], rc, sem1)
        cp1.wait()
        cp2.wait()

        def grp_body(k, cy):
            for l in range(L):
                e = k * L + l
                acc = jnp.zeros((L,), jnp.float32)
                for j in range(H // L):
                    a = rr[e, pl.ds(j * L, L)]
                    b = rc[e, pl.ds(j * L, L)]
                    d = a - b
                    acc = acc + d * d
                sred[pl.ds(0, L)] = acc
                v = sred[pl.ds(0, L)] + sred[pl.ds(8, L)]
                sred[pl.ds(0, L)] = v
                v = v + sred[pl.ds(4, L)]
                sred[pl.ds(0, L)] = v
                v = v + sred[pl.ds(2, L)]
                sred[pl.ds(0, L)] = v
                v = v + sred[pl.ds(1, L)]
                pack[pl.ds(l, L)] = v
            t = pack[pl.ds(0, L)] + 1e-12
            g = t * _newton_rsqrt(t)
            sg = 1.0 / (1.0 + jnp.exp(g))
            ecv = jnp.exp(sg)
            ec[pl.ds(k * L, L)] = ecv
            ed[pl.ds(k * L, L)] = E1 / ecv
            return cy

        lax.fori_loop(0, CHUNK // L, grp_body, 0)
        pltpu.sync_copy(ec, es_out.at[0, pl.ds(base, CHUNK)])
        pltpu.sync_copy(ed, es_out.at[1, pl.ds(base, CHUNK)])
        pltpu.sync_copy(ec, dcon_sh.at[ri], add=True)
        pltpu.sync_copy(ed, ddis_sh.at[ri], add=True)
        return carry

    lax.fori_loop(0, CPW, chunk_body, 0)
    plsc.subcore_barrier()
    pltpu.sync_copy(dcon_sh.at[pl.ds(s * NPT, NPT)], dstage)
    pltpu.sync_copy(dstage, dp_out.at[c, 0, pl.ds(s * NPT, NPT)])
    pltpu.sync_copy(ddis_sh.at[pl.ds(s * NPT, NPT)], dstage)
    pltpu.sync_copy(dstage, dp_out.at[c, 1, pl.ds(s * NPT, NPT)])


_phase1 = functools.partial(
    pl.kernel,
    mesh=plsc.VectorSubcoreMesh(core_axis_name="c", subcore_axis_name="s"),
    out_type=(
        jax.ShapeDtypeStruct((2, EPAD), jnp.float32),
        jax.ShapeDtypeStruct((NCORE, 2, NPAD), jnp.float32),
    ),
    scratch_types=[
        pltpu.VMEM((CHUNK,), jnp.int32),
        pltpu.VMEM((CHUNK,), jnp.int32),
        pltpu.VMEM((CHUNK, H), jnp.float32),
        pltpu.VMEM((CHUNK, H), jnp.float32),
        pltpu.VMEM((CHUNK,), jnp.float32),
        pltpu.VMEM((CHUNK,), jnp.float32),
        pltpu.VMEM((32,), jnp.float32),
        pltpu.VMEM((48,), jnp.float32),
        pltpu.VMEM((NPT,), jnp.float32),
        pltpu.VMEM_SHARED((NPAD,), jnp.float32),
        pltpu.VMEM_SHARED((NPAD,), jnp.float32),
        pltpu.SemaphoreType.DMA,
        pltpu.SemaphoreType.DMA,
    ],
)(_phase1_body)


def _norm_body(xc_ref, xd_ref, dp_ref, oc_ref, od_ref):
    dcol = dp_ref[...]
    dcon = dcol[:, 0:1] + dcol[:, 2:3] + 1e-16
    ddis = dcol[:, 1:2] + dcol[:, 3:4] + 1e-16
    oc_ref[...] = xc_ref[...] / dcon
    od_ref[...] = xd_ref[...] / ddis


def _norm(xc, xd, dp_t):
    n = xc.shape[0]
    br = 1024
    blk = lambda c: pl.BlockSpec((br, c), lambda i: (i, 0))
    return pl.pallas_call(
        _norm_body,
        grid=(n // br,),
        in_specs=[blk(H), blk(H), blk(4)],
        out_specs=[blk(H), blk(H)],
        out_shape=(jax.ShapeDtypeStruct((n, H), jnp.float32),
                   jax.ShapeDtypeStruct((n, H), jnp.float32)),
    )(xc, xd, dp_t)


def _phase2_body(xall, row, col, esw, out2,
                 ri0, ci0, rows0, ewb0, ri1, ci1, rows1, ewb1,
                 oacc, sem0, sem1):
    c = lax.axis_index("c")
    s = lax.axis_index("s")

    def zrow(r, cy):
        for j in range(H // L):
            rows0[r, pl.ds(j * L, L)] = jnp.zeros((L,), jnp.float32)
        return cy

    lax.fori_loop(0, CH2, zrow, 0)
    for b in range(NPT // CH2):
        pltpu.sync_copy(rows0, oacc.at[pl.ds(s * NPT + b * CH2, CH2)])
    plsc.subcore_barrier()

    def load_idx(ri, ci, ewb, base):
        pltpu.sync_copy(row.at[pl.ds(base, CH2)], ri)
        pltpu.sync_copy(col.at[pl.ds(base, CH2)], ci)
        pltpu.sync_copy(esw.at[c, pl.ds(base, CH2)], ewb)

        def offs(k, cy):
            ri[pl.ds(k * L, L)] = ri[pl.ds(k * L, L)] + c * NPAD
            return cy

        lax.fori_loop(0, CH2 // L, offs, 0)

    def do_chunk(ci, rows, ewb):
        def scale_body(e, cy):
            w = ewb[e, pl.ds(0, L)]
            for j in range(H // L):
                rows[e, pl.ds(j * L, L)] = rows[e, pl.ds(j * L, L)] * w
            return cy

        lax.fori_loop(0, CH2, scale_body, 0)
        pltpu.sync_copy(rows, oacc.at[ci], add=True)

    base0 = s * CPS2 * CH2
    load_idx(ri0, ci0, ewb0, base0)
    pltpu.async_copy(xall.at[ri0], rows0, sem0)

    def pair_body(p, carry):
        b1 = (s * CPS2 + 2 * p + 1) * CH2
        load_idx(ri1, ci1, ewb1, b1)
        pltpu.async_copy(xall.at[ri1], rows1, sem1)
        pltpu.make_async_copy(xall.at[ri0], rows0, sem0).wait()
        do_chunk(ci0, rows0, ewb0)

        @pl.when(p < CPS2 // 2 - 1)
        def _():
            b2 = (s * CPS2 + 2 * p + 2) * CH2
            load_idx(ri0, ci0, ewb0, b2)
            pltpu.async_copy(xall.at[ri0], rows0, sem0)

        pltpu.make_async_copy(xall.at[ri1], rows1, sem1).wait()
        do_chunk(ci1, rows1, ewb1)
        return carry

    lax.fori_loop(0, CPS2 // 2, pair_body, 0)
    plsc.subcore_barrier()
    for b in range(NPT // CH2):
        r0 = s * NPT + b * CH2
        pltpu.sync_copy(oacc.at[pl.ds(r0, CH2)], rows0)
        pltpu.sync_copy(rows0, out2.at[c, pl.ds(r0, CH2)])


_phase2 = functools.partial(
    pl.kernel,
    mesh=plsc.VectorSubcoreMesh(core_axis_name="c", subcore_axis_name="s"),
    out_type=jax.ShapeDtypeStruct((NCORE, NPAD, H), jnp.float32),
    scratch_types=[
        pltpu.VMEM((CH2,), jnp.int32),
        pltpu.VMEM((CH2,), jnp.int32),
        pltpu.VMEM((CH2, H), jnp.float32),
        pltpu.VMEM((CH2, L), jnp.float32),
        pltpu.VMEM((CH2,), jnp.int32),
        pltpu.VMEM((CH2,), jnp.int32),
        pltpu.VMEM((CH2, H), jnp.float32),
        pltpu.VMEM((CH2, L), jnp.float32),
        pltpu.VMEM_SHARED((NPAD, H), jnp.float32),
        pltpu.SemaphoreType.DMA,
        pltpu.SemaphoreType.DMA,
    ],
)(_phase2_body)


def kernel(x, edge_index, mlp_W, mlp_b, mlp_bn_g, mlp_bn_b,
           Wg0, Wcon0, Wdis0, Wself0, bself0, gW0, gb0,
           bn0_g, bn0_b,
           Wg1, Wcon1, Wdis1, Wself1, bself1, gW1, gb1,
           cls_W, cls_b):
    f32 = jnp.float32
    x_pad = jnp.zeros((NPAD, D), f32).at[:N].set(x)
    sl = jnp.arange(N, dtype=jnp.int32)
    pad_e = jnp.full((EPAD - ETOT,), N, jnp.int32)
    rowp = jnp.concatenate([edge_index[0].astype(jnp.int32), sl, pad_e])
    colp = jnp.concatenate([edge_index[1].astype(jnp.int32), sl, pad_e])

    ones_h = jnp.ones((1, H), f32)
    zeros_h = jnp.zeros((1, H), f32)
    bn_sc = (mlp_bn_g / jnp.sqrt(1.0 + EPS)).reshape(1, H)
    h = _lin(x_pad, mlp_W.T, mlp_b.reshape(1, H), bn_sc,
             mlp_bn_b.reshape(1, H), True)

    def conv_layer(hh, Wg, Wcon, Wdis, Wself, bself, gW, gb, bs, bt, bnrelu):
        wall = jnp.concatenate([Wg.T, Wcon.T, Wdis.T, Wself.T], axis=1)
        ball = jnp.concatenate([jnp.zeros((3 * H,), f32), bself]).reshape(1, 4 * H)
        on4 = jnp.ones((1, 4 * H), f32)
        ze4 = jnp.zeros((1, 4 * H), f32)
        y4 = _lin(hh, wall, ball, on4, ze4, False)
        xg = y4[:, 0:H]
        xcon = y4[:, H:2 * H]
        xdis = y4[:, 2 * H:3 * H]
        osf = y4[:, 3 * H:4 * H]
        es_arr, dparts = _phase1(xg, rowp, colp)
        dp_t = jnp.moveaxis(dparts, 2, 0).reshape(NPAD, 4)
        xcp, xdp = _norm(xcon, xdis, dp_t)
        esw = jnp.broadcast_to(es_arr[:, :, None], (2, EPAD, L))
        xall = jnp.concatenate([xcp, xdp], axis=0)
        out2 = _phase2(xall, rowp, colp, esw)
        gb128 = jnp.zeros((1, H), f32).at[0, :3].set(gb)
        return _tail(out2[0], out2[1], osf, hh, gW, gb128, bs, bt, bnrelu)

    bn0_sc = (bn0_g / jnp.sqrt(1.0 + EPS)).reshape(1, H)
    h = conv_layer(h, Wg0, Wcon0, Wdis0, Wself0, bself0, gW0, gb0,
                   bn0_sc, bn0_b.reshape(1, H), True)
    h = conv_layer(h, Wg1, Wcon1, Wdis1, Wself1, bself1, gW1, gb1,
                   ones_h, zeros_h, False)

    ones_c = jnp.ones((1, C), f32)
    zeros_c = jnp.zeros((1, C), f32)
    logits = _lin(h, cls_W.T, cls_b.reshape(1, C), ones_c, zeros_c, False)
    return logits[:N]


# speedup vs baseline: 8.3949x; 1.1152x over previous
import functools

import jax
import jax.numpy as jnp
from jax import lax
from jax.experimental import pallas as pl
from jax.experimental.pallas import tpu as pltpu
from jax.experimental.pallas import tpu_sc as plsc

N = 10000
D = 128
H = 128
C = 64
EPS = 1e-5
E1 = 2.718281828459045

NPAD = 10240
L = 16
NCORE = 2
NSUB = 16
NWORK = NCORE * NSUB
CHUNK = 128
ETOT = 320000 + N
CPW = -(-ETOT // (NWORK * CHUNK))
EPAD = NWORK * CPW * CHUNK
CH2 = 64
CPS2 = EPAD // (NSUB * CH2)
NPT = NPAD // NSUB


def _lin_body(x_ref, w_ref, b_ref, s_ref, t_ref, o_ref, *, relu):
    y = jnp.dot(x_ref[...], w_ref[...], preferred_element_type=jnp.float32)
    y = (y + b_ref[...]) * s_ref[...] + t_ref[...]
    if relu:
        y = jnp.maximum(y, 0.0)
    o_ref[...] = y


def _lin(x, w, b, s, t, relu):
    n, k = x.shape
    m = w.shape[1]
    br = 1024
    return pl.pallas_call(
        functools.partial(_lin_body, relu=relu),
        grid=(n // br,),
        in_specs=[
            pl.BlockSpec((br, k), lambda i: (i, 0)),
            pl.BlockSpec((k, m), lambda i: (0, 0)),
            pl.BlockSpec((1, m), lambda i: (0, 0)),
            pl.BlockSpec((1, m), lambda i: (0, 0)),
            pl.BlockSpec((1, m), lambda i: (0, 0)),
        ],
        out_specs=pl.BlockSpec((br, m), lambda i: (i, 0)),
        out_shape=jax.ShapeDtypeStruct((n, m), jnp.float32),
    )(x, w, b, s, t)


def _tail_body(oc_ref, od_ref, osf_ref, hin_ref, gw_ref, gb_ref, bs_ref,
               bt_ref, o_ref, *, bnrelu):
    oc = oc_ref[...]
    od = od_ref[...]
    osf = osf_ref[...]

    def logit(k):
        a = gw_ref[k:k + 1, 0:H]
        b = gw_ref[k:k + 1, H:2 * H]
        cc = gw_ref[k:k + 1, 2 * H:3 * H]
        return (jnp.sum(oc * a + od * b + osf * cc, axis=1, keepdims=True)
                + gb_ref[0:1, k:k + 1])

    z0 = logit(0)
    z1 = logit(1)
    z2 = logit(2)
    m = jnp.maximum(jnp.maximum(z0, z1), z2)
    e0 = jnp.exp(z0 - m)
    e1 = jnp.exp(z1 - m)
    e2 = jnp.exp(z2 - m)
    den = e0 + e1 + e2
    y = (e0 * oc + e1 * od + e2 * osf) / den
    if bnrelu:
        y = jnp.maximum(y * bs_ref[...] + bt_ref[...], 0.0)
    o_ref[...] = y + hin_ref[...]


def _tail(oc, od, osf, hin, gw, gb128, bs, bt, bnrelu):
    n = oc.shape[0]
    br = 1024
    blk = lambda r, c: pl.BlockSpec((r, c), lambda i: (i, 0))
    fix = lambda r, c: pl.BlockSpec((r, c), lambda i: (0, 0))
    return pl.pallas_call(
        functools.partial(_tail_body, bnrelu=bnrelu),
        grid=(n // br,),
        in_specs=[
            blk(br, H), blk(br, H), blk(br, H), blk(br, H),
            fix(3, 3 * H), fix(1, H), fix(1, H), fix(1, H),
        ],
        out_specs=blk(br, H),
        out_shape=jax.ShapeDtypeStruct((n, H), jnp.float32),
    )(oc, od, osf, hin, gw, gb128, bs, bt)


def _newton_rsqrt(t):
    i32 = lax.bitcast_convert_type(t, jnp.int32)
    i32 = 0x5F3759DF - lax.shift_right_logical(i32, 1)
    y = lax.bitcast_convert_type(i32, jnp.float32)
    y = y * (1.5 - 0.5 * t * y * y)
    y = y * (1.5 - 0.5 * t * y * y)
    y = y * (1.5 - 0.5 * t * y * y)
    return y


def _phase1_body(xg, row, col, es_out, dp_out,
                 ri0, ci0, rr0, rc0, ri1, ci1, rr1, rc1,
                 ec, ed, sred, pack, dstage, dcon_sh, ddis_sh,
                 sem0, sem1):
    c = lax.axis_index("c")
    s = lax.axis_index("s")
    wid = c * NSUB + s
    z16 = jnp.zeros((L,), jnp.float32)
    for k in range(NPT // L):
        dstage[pl.ds(k * L, L)] = z16
    pltpu.sync_copy(dstage, dcon_sh.at[pl.ds(s * NPT, NPT)])
    pltpu.sync_copy(dstage, ddis_sh.at[pl.ds(s * NPT, NPT)])
    plsc.subcore_barrier()

    def load_idx(ri, ci, base):
        pltpu.sync_copy(row.at[pl.ds(base, CHUNK)], ri)
        pltpu.sync_copy(col.at[pl.ds(base, CHUNK)], ci)

    def issue(ri, ci, rr, rc, sem):
        pltpu.async_copy(xg.at[ri], rr, sem)
        pltpu.async_copy(xg.at[ci], rc, sem)

    def wait(ri, ci, rr, rc, sem):
        pltpu.make_async_copy(xg.at[ri], rr, sem).wait()
        pltpu.make_async_copy(xg.at[ci], rc, sem).wait()

    def compute(ri, ci, rr, rc, base):
        def grp_body(k, cy):
            for l in range(L):
                e = k * L + l
                acc = jnp.zeros((L,), jnp.float32)
                for j in range(H // L):
                    a = rr[e, pl.ds(j * L, L)]
                    b = rc[e, pl.ds(j * L, L)]
                    d = a - b
                    acc = acc + d * d
                sred[pl.ds(0, L)] = acc
                v = sred[pl.ds(0, L)] + sred[pl.ds(8, L)]
                sred[pl.ds(0, L)] = v
                v = v + sred[pl.ds(4, L)]
                sred[pl.ds(0, L)] = v
                v = v + sred[pl.ds(2, L)]
                sred[pl.ds(0, L)] = v
                v = v + sred[pl.ds(1, L)]
                pack[pl.ds(l, L)] = v
            t = pack[pl.ds(0, L)] + 1e-12
            g = t * _newton_rsqrt(t)
            sg = 1.0 / (1.0 + jnp.exp(g))
            ecv = jnp.exp(sg)
            ec[pl.ds(k * L, L)] = ecv
            ed[pl.ds(k * L, L)] = E1 / ecv
            return cy

        lax.fori_loop(0, CHUNK // L, grp_body, 0)
        pltpu.sync_copy(ec, es_out.at[0, pl.ds(base, CHUNK)])
        pltpu.sync_copy(ed, es_out.at[1, pl.ds(base, CHUNK)])
        pltpu.sync_copy(ec, dcon_sh.at[ri], add=True)
        pltpu.sync_copy(ed, ddis_sh.at[ri], add=True)

    cbase = lambda ch: (wid * CPW + ch) * CHUNK
    load_idx(ri0, ci0, cbase(0))
    issue(ri0, ci0, rr0, rc0, sem0)

    def pair_body(p, carry):
        load_idx(ri1, ci1, cbase(2 * p + 1))
        issue(ri1, ci1, rr1, rc1, sem1)
        wait(ri0, ci0, rr0, rc0, sem0)
        compute(ri0, ci0, rr0, rc0, cbase(2 * p))

        @pl.when(p < CPW // 2 - 1)
        def _():
            load_idx(ri0, ci0, cbase(2 * p + 2))
            issue(ri0, ci0, rr0, rc0, sem0)

        wait(ri1, ci1, rr1, rc1, sem1)
        compute(ri1, ci1, rr1, rc1, cbase(2 * p + 1))
        return carry

    lax.fori_loop(0, CPW // 2, pair_body, 0)
    load_idx(ri0, ci0, cbase(CPW - 1))
    issue(ri0, ci0, rr0, rc0, sem0)
    wait(ri0, ci0, rr0, rc0, sem0)
    compute(ri0, ci0, rr0, rc0, cbase(CPW - 1))

    plsc.subcore_barrier()
    pltpu.sync_copy(dcon_sh.at[pl.ds(s * NPT, NPT)], dstage)
    pltpu.sync_copy(dstage, dp_out.at[c, 0, pl.ds(s * NPT, NPT)])
    pltpu.sync_copy(ddis_sh.at[pl.ds(s * NPT, NPT)], dstage)
    pltpu.sync_copy(dstage, dp_out.at[c, 1, pl.ds(s * NPT, NPT)])


_phase1 = functools.partial(
    pl.kernel,
    mesh=plsc.VectorSubcoreMesh(core_axis_name="c", subcore_axis_name="s"),
    out_type=(
        jax.ShapeDtypeStruct((2, EPAD), jnp.float32),
        jax.ShapeDtypeStruct((NCORE, 2, NPAD), jnp.float32),
    ),
    scratch_types=[
        pltpu.VMEM((CHUNK,), jnp.int32),
        pltpu.VMEM((CHUNK,), jnp.int32),
        pltpu.VMEM((CHUNK, H), jnp.float32),
        pltpu.VMEM((CHUNK, H), jnp.float32),
        pltpu.VMEM((CHUNK,), jnp.int32),
        pltpu.VMEM((CHUNK,), jnp.int32),
        pltpu.VMEM((CHUNK, H), jnp.float32),
        pltpu.VMEM((CHUNK, H), jnp.float32),
        pltpu.VMEM((CHUNK,), jnp.float32),
        pltpu.VMEM((CHUNK,), jnp.float32),
        pltpu.VMEM((32,), jnp.float32),
        pltpu.VMEM((48,), jnp.float32),
        pltpu.VMEM((NPT,), jnp.float32),
        pltpu.VMEM_SHARED((NPAD,), jnp.float32),
        pltpu.VMEM_SHARED((NPAD,), jnp.float32),
        pltpu.SemaphoreType.DMA,
        pltpu.SemaphoreType.DMA,
    ],
)(_phase1_body)


def _norm_body(xc_ref, xd_ref, dp_ref, oc_ref, od_ref):
    dcol = dp_ref[...]
    dcon = dcol[:, 0:1] + dcol[:, 2:3] + 1e-16
    ddis = dcol[:, 1:2] + dcol[:, 3:4] + 1e-16
    oc_ref[...] = xc_ref[...] / dcon
    od_ref[...] = xd_ref[...] / ddis


def _norm(xc, xd, dp_t):
    n = xc.shape[0]
    br = 1024
    blk = lambda c: pl.BlockSpec((br, c), lambda i: (i, 0))
    return pl.pallas_call(
        _norm_body,
        grid=(n // br,),
        in_specs=[blk(H), blk(H), blk(4)],
        out_specs=[blk(H), blk(H)],
        out_shape=(jax.ShapeDtypeStruct((n, H), jnp.float32),
                   jax.ShapeDtypeStruct((n, H), jnp.float32)),
    )(xc, xd, dp_t)


def _phase2_body(xall, row, col, esw, out2,
                 ri0, ci0, rows0, ewb0, ri1, ci1, rows1, ewb1,
                 oacc, sem0, sem1):
    c = lax.axis_index("c")
    s = lax.axis_index("s")

    def zrow(r, cy):
        for j in range(H // L):
            rows0[r, pl.ds(j * L, L)] = jnp.zeros((L,), jnp.float32)
        return cy

    lax.fori_loop(0, CH2, zrow, 0)
    for b in range(NPT // CH2):
        pltpu.sync_copy(rows0, oacc.at[pl.ds(s * NPT + b * CH2, CH2)])
    plsc.subcore_barrier()

    def load_idx(ri, ci, ewb, base):
        pltpu.sync_copy(row.at[pl.ds(base, CH2)], ri)
        pltpu.sync_copy(col.at[pl.ds(base, CH2)], ci)
        pltpu.sync_copy(esw.at[c, pl.ds(base, CH2)], ewb)

        def offs(k, cy):
            ri[pl.ds(k * L, L)] = ri[pl.ds(k * L, L)] + c * NPAD
            return cy

        lax.fori_loop(0, CH2 // L, offs, 0)

    def do_chunk(ci, rows, ewb):
        def scale_body(e, cy):
            w = ewb[e, pl.ds(0, L)]
            for j in range(H // L):
                rows[e, pl.ds(j * L, L)] = rows[e, pl.ds(j * L, L)] * w
            return cy

        lax.fori_loop(0, CH2, scale_body, 0)
        pltpu.sync_copy(rows, oacc.at[ci], add=True)

    base0 = s * CPS2 * CH2
    load_idx(ri0, ci0, ewb0, base0)
    pltpu.async_copy(xall.at[ri0], rows0, sem0)

    def pair_body(p, carry):
        b1 = (s * CPS2 + 2 * p + 1) * CH2
        load_idx(ri1, ci1, ewb1, b1)
        pltpu.async_copy(xall.at[ri1], rows1, sem1)
        pltpu.make_async_copy(xall.at[ri0], rows0, sem0).wait()
        do_chunk(ci0, rows0, ewb0)

        @pl.when(p < CPS2 // 2 - 1)
        def _():
            b2 = (s * CPS2 + 2 * p + 2) * CH2
            load_idx(ri0, ci0, ewb0, b2)
            pltpu.async_copy(xall.at[ri0], rows0, sem0)

        pltpu.make_async_copy(xall.at[ri1], rows1, sem1).wait()
        do_chunk(ci1, rows1, ewb1)
        return carry

    lax.fori_loop(0, CPS2 // 2, pair_body, 0)
    plsc.subcore_barrier()
    for b in range(NPT // CH2):
        r0 = s * NPT + b * CH2
        pltpu.sync_copy(oacc.at[pl.ds(r0, CH2)], rows0)
        pltpu.sync_copy(rows0, out2.at[c, pl.ds(r0, CH2)])


_phase2 = functools.partial(
    pl.kernel,
    mesh=plsc.VectorSubcoreMesh(core_axis_name="c", subcore_axis_name="s"),
    out_type=jax.ShapeDtypeStruct((NCORE, NPAD, H), jnp.float32),
    scratch_types=[
        pltpu.VMEM((CH2,), jnp.int32),
        pltpu.VMEM((CH2,), jnp.int32),
        pltpu.VMEM((CH2, H), jnp.float32),
        pltpu.VMEM((CH2, L), jnp.float32),
        pltpu.VMEM((CH2,), jnp.int32),
        pltpu.VMEM((CH2,), jnp.int32),
        pltpu.VMEM((CH2, H), jnp.float32),
        pltpu.VMEM((CH2, L), jnp.float32),
        pltpu.VMEM_SHARED((NPAD, H), jnp.float32),
        pltpu.SemaphoreType.DMA,
        pltpu.SemaphoreType.DMA,
    ],
)(_phase2_body)


def kernel(x, edge_index, mlp_W, mlp_b, mlp_bn_g, mlp_bn_b,
           Wg0, Wcon0, Wdis0, Wself0, bself0, gW0, gb0,
           bn0_g, bn0_b,
           Wg1, Wcon1, Wdis1, Wself1, bself1, gW1, gb1,
           cls_W, cls_b):
    f32 = jnp.float32
    x_pad = jnp.zeros((NPAD, D), f32).at[:N].set(x)
    sl = jnp.arange(N, dtype=jnp.int32)
    pad_e = jnp.full((EPAD - ETOT,), N, jnp.int32)
    rowp = jnp.concatenate([edge_index[0].astype(jnp.int32), sl, pad_e])
    colp = jnp.concatenate([edge_index[1].astype(jnp.int32), sl, pad_e])

    ones_h = jnp.ones((1, H), f32)
    zeros_h = jnp.zeros((1, H), f32)
    bn_sc = (mlp_bn_g / jnp.sqrt(1.0 + EPS)).reshape(1, H)
    h = _lin(x_pad, mlp_W.T, mlp_b.reshape(1, H), bn_sc,
             mlp_bn_b.reshape(1, H), True)

    def conv_layer(hh, Wg, Wcon, Wdis, Wself, bself, gW, gb, bs, bt, bnrelu):
        wall = jnp.concatenate([Wg.T, Wcon.T, Wdis.T, Wself.T], axis=1)
        ball = jnp.concatenate([jnp.zeros((3 * H,), f32), bself]).reshape(1, 4 * H)
        on4 = jnp.ones((1, 4 * H), f32)
        ze4 = jnp.zeros((1, 4 * H), f32)
        y4 = _lin(hh, wall, ball, on4, ze4, False)
        xg = y4[:, 0:H]
        xcon = y4[:, H:2 * H]
        xdis = y4[:, 2 * H:3 * H]
        osf = y4[:, 3 * H:4 * H]
        es_arr, dparts = _phase1(xg, rowp, colp)
        dp_t = jnp.moveaxis(dparts, 2, 0).reshape(NPAD, 4)
        xcp, xdp = _norm(xcon, xdis, dp_t)
        esw = jnp.broadcast_to(es_arr[:, :, None], (2, EPAD, L))
        xall = jnp.concatenate([xcp, xdp], axis=0)
        out2 = _phase2(xall, rowp, colp, esw)
        gb128 = jnp.zeros((1, H), f32).at[0, :3].set(gb)
        return _tail(out2[0], out2[1], osf, hh, gW, gb128, bs, bt, bnrelu)

    bn0_sc = (bn0_g / jnp.sqrt(1.0 + EPS)).reshape(1, H)
    h = conv_layer(h, Wg0, Wcon0, Wdis0, Wself0, bself0, gW0, gb0,
                   bn0_sc, bn0_b.reshape(1, H), True)
    h = conv_layer(h, Wg1, Wcon1, Wdis1, Wself1, bself1, gW1, gb1,
                   ones_h, zeros_h, False)

    ones_c = jnp.ones((1, C), f32)
    zeros_c = jnp.zeros((1, C), f32)
    logits = _lin(h, cls_W.T, cls_b.reshape(1, C), ones_c, zeros_c, False)
    return logits[:N]
